# baseline (speedup 1.0000x reference)
import functools
import math

import jax
import jax.numpy as jnp
from jax import lax
from jax.experimental import pallas as pl
from jax.experimental.pallas import tpu as pltpu
from jax.experimental.pallas import tpu_sc as plsc

_N, _S, _H, _F, _K = 10000, 18, 128, 64, 2048
_R = _N * _S
_NW = 32
_BPW = 5632
_CH = 128
_NCH = _BPW // _CH
_RPAD = _NW * _BPW
_BA = 400
_NB = _N // _BA


def _ln(x):
    mu = jnp.mean(x, axis=-1, keepdims=True)
    xc = x - mu
    var = jnp.mean(xc * xc, axis=-1, keepdims=True)
    return xc * lax.rsqrt(var + 1e-5)


def _ln_mm(x, u):
    mu = jnp.dot(x, u, preferred_element_type=jnp.float32)[:, 0:1]
    m2 = jnp.dot(x * x, u, preferred_element_type=jnp.float32)[:, 0:1]
    var = m2 - mu * mu
    return (x - mu) * lax.rsqrt(var + 1e-5)


_CC = (1.0, -19.739208, 64.93939, -85.45669, 60.242466, -26.406763,
       7.8066154, -1.4609568)
_SS = (6.2831855, -41.3417, 81.60525, -76.70578, 42.057533, -15.085474,
       3.7785523, -0.6179781)
_MAGIC = 12582912.0


def _cossin2pi(y):
    r = y - ((y + _MAGIC) - _MAGIC)
    r2 = r * r
    c = jnp.float32(_CC[-1])
    for a in _CC[-2::-1]:
        c = c * r2 + a
    s = jnp.float32(_SS[-1])
    for b in _SS[-2::-1]:
        s = s * r2 + b
    return c, s * r


def _dot(a, b):
    return jnp.dot(a.astype(jnp.bfloat16), b.astype(jnp.bfloat16),
                   preferred_element_type=jnp.float32)


def _tables_body(tok_ref, w1_ref, b1_ref, w2_ref, b2_ref, wtop_ref, out_ref):
    tok = tok_ref[0]
    h = jax.nn.relu(_ln(_dot(tok, w1_ref[0]) + b1_ref[0]))
    emb = _dot(h, w2_ref[0]) + b2_ref[0]
    out_ref[0] = _dot(emb, wtop_ref[...])


def _token_tables(tok3, w1s, b1s, w2s, b2s, wtop):
    out = pl.pallas_call(
        _tables_body,
        grid=(3,),
        in_specs=[
            pl.BlockSpec((1, _K, 8), lambda i: (i, 0, 0)),
            pl.BlockSpec((1, 8, _H), lambda i: (i, 0, 0)),
            pl.BlockSpec((1, 1, _H), lambda i: (i, 0, 0)),
            pl.BlockSpec((1, _H, _H), lambda i: (i, 0, 0)),
            pl.BlockSpec((1, 1, _H), lambda i: (i, 0, 0)),
            pl.BlockSpec((_H, _H), lambda i: (0, 0)),
        ],
        out_specs=pl.BlockSpec((1, _K, _H), lambda i: (i, 0, 0)),
        out_shape=jax.ShapeDtypeStruct((3, _K, _H), jnp.float32),
    )(tok3, w1s, b1s, w2s, b2s, wtop)
    return out.reshape(3 * _K, _H)


def _sc_gather(table, idx_pad):
    mesh = plsc.VectorSubcoreMesh(core_axis_name="c", subcore_axis_name="s")

    @functools.partial(
        pl.kernel,
        mesh=mesh,
        out_type=jax.ShapeDtypeStruct((_R, _H), jnp.float32),
        scratch_types=[
            pltpu.VMEM((_BPW,), jnp.int32),
            pltpu.VMEM((_CH, _H), jnp.float32),
            pltpu.SemaphoreType.DMA,
        ],
    )
    def gk(table_hbm, idx_hbm, out_hbm, idx_v, rows_v, sem):
        wid = lax.axis_index("s") * 2 + lax.axis_index("c")
        base = wid * _BPW
        pltpu.sync_copy(idx_hbm.at[pl.ds(base, _BPW)], idx_v)

        def chunk(j, carry):
            start = base + j * _CH

            @pl.when(start < _R)
            def _():
                pltpu.async_copy(
                    table_hbm.at[idx_v.at[pl.ds(j * _CH, _CH)]], rows_v, sem
                ).wait()

            @pl.when(start + _CH <= _R)
            def _():
                pltpu.sync_copy(rows_v, out_hbm.at[pl.ds(start, _CH)])

            @pl.when(jnp.logical_and(start < _R, start + _CH > _R))
            def _():
                pltpu.sync_copy(
                    rows_v.at[pl.ds(0, _R % _CH)],
                    out_hbm.at[pl.ds(start, _R % _CH)],
                )

            return carry

        lax.fori_loop(0, _NCH, chunk, 0)

    return gk(table, idx_pad)


_BR = _BA * _S


def _main_body(px_ref, py_ref, hx_ref, hy_ref, shp_ref, oh_ref, gc_ref,
               f0_ref, f1_ref, wc0_ref, ws0_ref, wx0_ref, b10_ref,
               wc1_ref, ws1_ref, wx1_ref, b11_ref,
               w20_ref, b20_ref, w21_ref, b21_ref,
               outw_ref, outb_ref,
               sw1_ref, sb1_ref, sw2_ref, sb2_ref, te_ref,
               fwb_ref, fb1_ref, fw2_ref, fb2_ref, u_ref,
               o_ref):
    px = px_ref[...]
    py = py_ref[...]
    mx = px - jnp.concatenate([px[:, :1], px[:, : _S - 1]], axis=1)
    my = py - jnp.concatenate([py[:, :1], py[:, : _S - 1]], axis=1)
    hx = hx_ref[...]
    hy = hy_ref[...]
    nrm = jnp.sqrt(mx * mx + my * my)
    ang = jnp.arctan2(hx * my - hy * mx, hx * mx + hy * my)
    ang = jnp.concatenate([jnp.zeros((_BA, 1), jnp.float32), ang[:, 1:]], axis=1)

    f0 = f0_ref[...]
    f1 = f1_ref[...]
    uu = u_ref[...]
    wc0, ws0, wx0 = wc0_ref[...], ws0_ref[...], wx0_ref[...]
    wc1, ws1, wx1 = wc1_ref[...], ws1_ref[...], wx1_ref[...]
    p0s, p1s = [], []
    for s in range(_S):
        nc = nrm[:, s : s + 1]
        ac = ang[:, s : s + 1]
        c0, s0 = _cossin2pi(nc * f0)
        c1, s1 = _cossin2pi(ac * f1)
        p0s.append(_dot(c0, wc0) + _dot(s0, ws0) + nc * wx0)
        p1s.append(_dot(c1, wc1) + _dot(s1, ws1) + ac * wx1)
    pre0 = jnp.concatenate(p0s, axis=0) + b10_ref[...]
    pre1 = jnp.concatenate(p1s, axis=0) + b11_ref[...]
    h0 = _dot(jax.nn.relu(_ln_mm(pre0, uu)), w20_ref[...]) + b20_ref[...]
    h1 = _dot(jax.nn.relu(_ln_mm(pre1, uu)), w21_ref[...]) + b21_ref[...]
    cat = _dot(
        jax.nn.relu(_ln_mm(_dot(shp_ref[...], sw1_ref[...]) + sb1_ref[...], uu)),
        sw2_ref[...],
    ) + sb2_ref[...] + _dot(oh_ref[...], te_ref[...])
    catr = jnp.concatenate([cat] * _S, axis=0)
    xa = _dot(jax.nn.relu(_ln_mm(h0 + h1 + catr, uu)), outw_ref[...]) + outb_ref[...]
    pre = gc_ref[...] + _dot(xa, fwb_ref[...]) + fb1_ref[...]
    feat = _dot(jax.nn.relu(_ln_mm(pre, uu)), fw2_ref[...]) + fb2_ref[...]
    for s in range(_S):
        o_ref[:, s, :] = feat[s * _BA : (s + 1) * _BA, :]


def _main(px, py, hx, hy, shp, oh, gc, weights):
    row = lambda: pl.BlockSpec((_BA, _S), lambda i: (i, 0))
    small = lambda a: pl.BlockSpec(a.shape, lambda i: (0,) * a.ndim)
    return pl.pallas_call(
        _main_body,
        grid=(_NB,),
        in_specs=[
            row(), row(), row(), row(),
            pl.BlockSpec((_BA, 3), lambda i: (i, 0)),
            pl.BlockSpec((_BA, 3), lambda i: (i, 0)),
            pl.BlockSpec((_BR, _H), lambda i: (i, 0)),
        ] + [small(w) for w in weights],
        out_specs=pl.BlockSpec((_BA, _S, _H), lambda i: (i, 0, 0)),
        out_shape=jax.ShapeDtypeStruct((_N, _S, _H), jnp.float32),
    )(px, py, hx, hy, shp, oh, gc, *weights)


def kernel(agent_token_index, trajectory_token_veh, trajectory_token_ped,
           trajectory_token_cyc, pos_a, head_vector_a, agent_type, agent_shape,
           type_emb_w, shape_W1, shape_b1, shape_W2, shape_b2,
           veh_W1, veh_b1, veh_W2, veh_b2, ped_W1, ped_b1, ped_W2, ped_b2,
           cyc_W1, cyc_b1, cyc_W2, cyc_b2, freqs, xa_W1, xa_b1, xa_W2, xa_b2,
           out_W, out_b, fus_W1, fus_b1, fus_W2, fus_b2):
    tok3 = jnp.stack([trajectory_token_veh, trajectory_token_ped,
                      trajectory_token_cyc])
    w1s = jnp.stack([veh_W1, ped_W1, cyc_W1])
    b1s = jnp.stack([veh_b1, ped_b1, cyc_b1])[:, None, :]
    w2s = jnp.stack([veh_W2, ped_W2, cyc_W2])
    b2s = jnp.stack([veh_b2, ped_b2, cyc_b2])[:, None, :]
    table = _token_tables(tok3, w1s, b1s, w2s, b2s, fus_W1[:_H])

    atype = agent_type.astype(jnp.int32)
    idx = (atype[:, None] * _K + agent_token_index.astype(jnp.int32)).reshape(-1)
    ii = jnp.arange(_NB, dtype=jnp.int32)[:, None, None]
    ss = jnp.arange(_S, dtype=jnp.int32)[None, :, None]
    aa = jnp.arange(_BA, dtype=jnp.int32)[None, None, :]
    orig = ((ii * _BA + aa) * _S + ss).reshape(-1)
    idx_pad = jnp.pad(idx[orig], (0, _RPAD - _R))
    gathered = _sc_gather(table, idx_pad)

    weights = (
        freqs[0:1],
        freqs[1:2],
        xa_W1[0, :_F, :], xa_W1[0, _F:2 * _F, :], xa_W1[0, 2 * _F:, :],
        xa_b1[0:1],
        xa_W1[1, :_F, :], xa_W1[1, _F:2 * _F, :], xa_W1[1, 2 * _F:, :],
        xa_b1[1:2],
        xa_W2[0], xa_b2[0:1], xa_W2[1], xa_b2[1:2],
        out_W, out_b[None, :],
        shape_W1, shape_b1[None, :], shape_W2, shape_b2[None, :], type_emb_w,
        fus_W1[_H:], fus_b1[None, :], fus_W2, fus_b2[None, :],
        jnp.zeros((_H, 8), jnp.float32).at[:, 0].set(1.0 / _H),
    )
    oh = jax.nn.one_hot(atype, 3, dtype=jnp.float32)
    return _main(pos_a[..., 0], pos_a[..., 1],
                 head_vector_a[..., 0], head_vector_a[..., 1],
                 agent_shape, oh, gathered, weights)


# speedup vs baseline: 1.0874x; 1.0874x over previous
import functools
import math

import jax
import jax.numpy as jnp
from jax import lax
from jax.experimental import pallas as pl
from jax.experimental.pallas import tpu as pltpu
from jax.experimental.pallas import tpu_sc as plsc

_N, _S, _H, _F, _K = 10000, 18, 128, 64, 2048
_R = _N * _S
_NW = 32
_BPW = 5632
_CH = 128
_NCH = _BPW // _CH
_RPAD = _NW * _BPW
_BA = 400
_NB = _N // _BA


def _ln(x):
    mu = jnp.mean(x, axis=-1, keepdims=True)
    xc = x - mu
    var = jnp.mean(xc * xc, axis=-1, keepdims=True)
    return xc * lax.rsqrt(var + 1e-5)


_CC = (1.0, -19.739208, 64.93939, -85.45669, 60.242466, -26.406763,
       7.8066154, -1.4609568)
_SS = (6.2831855, -41.3417, 81.60525, -76.70578, 42.057533, -15.085474,
       3.7785523, -0.6179781)
_MAGIC = 12582912.0


def _cossin2pi(y):
    r = y - ((y + _MAGIC) - _MAGIC)
    r2 = r * r
    c = jnp.float32(_CC[-1])
    for a in _CC[-2::-1]:
        c = c * r2 + a
    s = jnp.float32(_SS[-1])
    for b in _SS[-2::-1]:
        s = s * r2 + b
    return c, s * r


def _dot(a, b):
    return jnp.dot(a.astype(jnp.bfloat16), b.astype(jnp.bfloat16),
                   preferred_element_type=jnp.float32)


def _tables_body(tok_ref, w1_ref, b1_ref, w2_ref, b2_ref, wtop_ref, out_ref):
    tok = tok_ref[0]
    h = jax.nn.relu(_ln(_dot(tok, w1_ref[0]) + b1_ref[0]))
    emb = _dot(h, w2_ref[0]) + b2_ref[0]
    out_ref[0] = _dot(emb, wtop_ref[...])


def _token_tables(tok3, w1s, b1s, w2s, b2s, wtop):
    out = pl.pallas_call(
        _tables_body,
        grid=(3,),
        in_specs=[
            pl.BlockSpec((1, _K, 8), lambda i: (i, 0, 0)),
            pl.BlockSpec((1, 8, _H), lambda i: (i, 0, 0)),
            pl.BlockSpec((1, 1, _H), lambda i: (i, 0, 0)),
            pl.BlockSpec((1, _H, _H), lambda i: (i, 0, 0)),
            pl.BlockSpec((1, 1, _H), lambda i: (i, 0, 0)),
            pl.BlockSpec((_H, _H), lambda i: (0, 0)),
        ],
        out_specs=pl.BlockSpec((1, _K, _H), lambda i: (i, 0, 0)),
        out_shape=jax.ShapeDtypeStruct((3, _K, _H), jnp.float32),
    )(tok3, w1s, b1s, w2s, b2s, wtop)
    return out.reshape(3 * _K, _H)


def _sc_gather(table, idx_pad):
    mesh = plsc.VectorSubcoreMesh(core_axis_name="c", subcore_axis_name="s")

    @functools.partial(
        pl.kernel,
        mesh=mesh,
        out_type=jax.ShapeDtypeStruct((_R, _H), jnp.float32),
        scratch_types=[
            pltpu.VMEM((_BPW,), jnp.int32),
            pltpu.VMEM((2, _CH, _H), jnp.float32),
            pltpu.SemaphoreType.DMA,
            pltpu.SemaphoreType.DMA,
        ],
    )
    def gk(table_hbm, idx_hbm, out_hbm, idx_v, rows_v, sem0, sem1):
        wid = lax.axis_index("s") * 2 + lax.axis_index("c")
        base = wid * _BPW
        pltpu.sync_copy(idx_hbm.at[pl.ds(base, _BPW)], idx_v)

        def live(j):
            return jnp.logical_and(j < _NCH, base + j * _CH < _R)

        def start(j, slot, sem):
            @pl.when(live(j))
            def _():
                pltpu.async_copy(
                    table_hbm.at[idx_v.at[pl.ds(j * _CH, _CH)]],
                    rows_v.at[slot], sem)

        def wait(j, slot, sem):
            @pl.when(live(j))
            def _():
                pltpu.make_async_copy(
                    table_hbm.at[idx_v.at[pl.ds(j * _CH, _CH)]],
                    rows_v.at[slot], sem).wait()

        def drain(j, slot):
            row0 = base + j * _CH

            @pl.when(row0 + _CH <= _R)
            def _():
                pltpu.sync_copy(rows_v.at[slot], out_hbm.at[pl.ds(row0, _CH)])

            @pl.when(jnp.logical_and(row0 < _R, row0 + _CH > _R))
            def _():
                pltpu.sync_copy(
                    rows_v.at[slot].at[pl.ds(0, _R % _CH)],
                    out_hbm.at[pl.ds(row0, _R % _CH)],
                )

        start(jnp.int32(0), 0, sem0)

        def body2(i, carry):
            j = i * 2
            start(j + 1, 1, sem1)
            wait(j, 0, sem0)
            drain(j, 0)
            start(j + 2, 0, sem0)
            wait(j + 1, 1, sem1)
            drain(j + 1, 1)
            return carry

        lax.fori_loop(0, _NCH // 2, body2, 0)

    return gk(table, idx_pad)


_BR = _BA * _S


def _main_body(px_ref, py_ref, hx_ref, hy_ref, shp_ref, oh_ref, gc_ref,
               f0_ref, f1_ref, wc0_ref, ws0_ref, wx0_ref, b10_ref,
               wc1_ref, ws1_ref, wx1_ref, b11_ref,
               w20_ref, b20_ref, w21_ref, b21_ref,
               outw_ref, outb_ref,
               sw1_ref, sb1_ref, sw2_ref, sb2_ref, te_ref,
               fwb_ref, fb1_ref, fw2_ref, fb2_ref,
               o_ref):
    px = px_ref[...]
    py = py_ref[...]
    mx = px - jnp.concatenate([px[:, :1], px[:, : _S - 1]], axis=1)
    my = py - jnp.concatenate([py[:, :1], py[:, : _S - 1]], axis=1)
    hx = hx_ref[...]
    hy = hy_ref[...]
    nrm = jnp.sqrt(mx * mx + my * my)
    ang = jnp.arctan2(hx * my - hy * mx, hx * mx + hy * my)
    ang = jnp.concatenate([jnp.zeros((_BA, 1), jnp.float32), ang[:, 1:]], axis=1)

    f0 = f0_ref[...]
    f1 = f1_ref[...]
    wc0, ws0, wx0 = wc0_ref[...], ws0_ref[...], wx0_ref[...]
    wc1, ws1, wx1 = wc1_ref[...], ws1_ref[...], wx1_ref[...]
    p0s, p1s = [], []
    for s in range(_S):
        nc = nrm[:, s : s + 1]
        ac = ang[:, s : s + 1]
        c0, s0 = _cossin2pi(nc * f0)
        c1, s1 = _cossin2pi(ac * f1)
        p0s.append(_dot(c0, wc0) + _dot(s0, ws0) + nc * wx0)
        p1s.append(_dot(c1, wc1) + _dot(s1, ws1) + ac * wx1)
    pre0 = jnp.concatenate(p0s, axis=0) + b10_ref[...]
    pre1 = jnp.concatenate(p1s, axis=0) + b11_ref[...]
    h0 = _dot(jax.nn.relu(_ln(pre0)), w20_ref[...]) + b20_ref[...]
    h1 = _dot(jax.nn.relu(_ln(pre1)), w21_ref[...]) + b21_ref[...]
    cat = _dot(
        jax.nn.relu(_ln(_dot(shp_ref[...], sw1_ref[...]) + sb1_ref[...])),
        sw2_ref[...],
    ) + sb2_ref[...] + _dot(oh_ref[...], te_ref[...])
    catr = jnp.concatenate([cat] * _S, axis=0)
    xa = _dot(jax.nn.relu(_ln(h0 + h1 + catr)), outw_ref[...]) + outb_ref[...]
    pre = gc_ref[...] + _dot(xa, fwb_ref[...]) + fb1_ref[...]
    feat = _dot(jax.nn.relu(_ln(pre)), fw2_ref[...]) + fb2_ref[...]
    for s in range(_S):
        o_ref[:, s, :] = feat[s * _BA : (s + 1) * _BA, :]


def _main(px, py, hx, hy, shp, oh, gc, weights):
    row = lambda: pl.BlockSpec((_BA, _S), lambda i: (i, 0))
    small = lambda a: pl.BlockSpec(a.shape, lambda i: (0,) * a.ndim)
    return pl.pallas_call(
        _main_body,
        grid=(_NB,),
        in_specs=[
            row(), row(), row(), row(),
            pl.BlockSpec((_BA, 3), lambda i: (i, 0)),
            pl.BlockSpec((_BA, 3), lambda i: (i, 0)),
            pl.BlockSpec((_BR, _H), lambda i: (i, 0)),
        ] + [small(w) for w in weights],
        out_specs=pl.BlockSpec((_BA, _S, _H), lambda i: (i, 0, 0)),
        out_shape=jax.ShapeDtypeStruct((_N, _S, _H), jnp.float32),
    )(px, py, hx, hy, shp, oh, gc, *weights)


def kernel(agent_token_index, trajectory_token_veh, trajectory_token_ped,
           trajectory_token_cyc, pos_a, head_vector_a, agent_type, agent_shape,
           type_emb_w, shape_W1, shape_b1, shape_W2, shape_b2,
           veh_W1, veh_b1, veh_W2, veh_b2, ped_W1, ped_b1, ped_W2, ped_b2,
           cyc_W1, cyc_b1, cyc_W2, cyc_b2, freqs, xa_W1, xa_b1, xa_W2, xa_b2,
           out_W, out_b, fus_W1, fus_b1, fus_W2, fus_b2):
    tok3 = jnp.stack([trajectory_token_veh, trajectory_token_ped,
                      trajectory_token_cyc])
    w1s = jnp.stack([veh_W1, ped_W1, cyc_W1])
    b1s = jnp.stack([veh_b1, ped_b1, cyc_b1])[:, None, :]
    w2s = jnp.stack([veh_W2, ped_W2, cyc_W2])
    b2s = jnp.stack([veh_b2, ped_b2, cyc_b2])[:, None, :]
    table = _token_tables(tok3, w1s, b1s, w2s, b2s, fus_W1[:_H])

    atype = agent_type.astype(jnp.int32)
    idx = (atype[:, None] * _K + agent_token_index.astype(jnp.int32)).reshape(-1)
    ii = jnp.arange(_NB, dtype=jnp.int32)[:, None, None]
    ss = jnp.arange(_S, dtype=jnp.int32)[None, :, None]
    aa = jnp.arange(_BA, dtype=jnp.int32)[None, None, :]
    orig = ((ii * _BA + aa) * _S + ss).reshape(-1)
    idx_pad = jnp.pad(idx[orig], (0, _RPAD - _R))
    gathered = _sc_gather(table, idx_pad)

    weights = (
        freqs[0:1],
        freqs[1:2],
        xa_W1[0, :_F, :], xa_W1[0, _F:2 * _F, :], xa_W1[0, 2 * _F:, :],
        xa_b1[0:1],
        xa_W1[1, :_F, :], xa_W1[1, _F:2 * _F, :], xa_W1[1, 2 * _F:, :],
        xa_b1[1:2],
        xa_W2[0], xa_b2[0:1], xa_W2[1], xa_b2[1:2],
        out_W, out_b[None, :],
        shape_W1, shape_b1[None, :], shape_W2, shape_b2[None, :], type_emb_w,
        fus_W1[_H:], fus_b1[None, :], fus_W2, fus_b2[None, :],
    )
    oh = jax.nn.one_hot(atype, 3, dtype=jnp.float32)
    return _main(pos_a[..., 0], pos_a[..., 1],
                 head_vector_a[..., 0], head_vector_a[..., 1],
                 agent_shape, oh, gathered, weights)


# speedup vs baseline: 1.0968x; 1.0087x over previous
import functools
import math

import jax
import jax.numpy as jnp
from jax import lax
from jax.experimental import pallas as pl
from jax.experimental.pallas import tpu as pltpu
from jax.experimental.pallas import tpu_sc as plsc

_N, _S, _H, _F, _K = 10000, 18, 128, 64, 2048
_R = _N * _S
_NW = 32
_BPW = 5632
_CH = 128
_NCH = _BPW // _CH
_RPAD = _NW * _BPW
_BA = 400
_NB = _N // _BA


def _ln(x):
    mu = jnp.mean(x, axis=-1, keepdims=True)
    xc = x - mu
    var = jnp.mean(xc * xc, axis=-1, keepdims=True)
    return xc * lax.rsqrt(var + 1e-5)


_CC = (1.0, -19.739208, 64.93939, -85.45669, 60.242466, -26.406763,
       7.8066154, -1.4609568)
_SS = (6.2831855, -41.3417, 81.60525, -76.70578, 42.057533, -15.085474,
       3.7785523, -0.6179781)
_MAGIC = 12582912.0


def _cossin2pi(y):
    r = y - jnp.round(y)
    r2 = r * r
    c = jnp.float32(_CC[-1])
    for a in _CC[-2::-1]:
        c = c * r2 + a
    s = jnp.float32(_SS[-1])
    for b in _SS[-2::-1]:
        s = s * r2 + b
    return c, s * r


def _dot(a, b):
    return jnp.dot(a.astype(jnp.bfloat16), b.astype(jnp.bfloat16),
                   preferred_element_type=jnp.float32)


def _tables_body(tok_ref, w1_ref, b1_ref, w2_ref, b2_ref, wtop_ref, out_ref):
    tok = tok_ref[0]
    h = jax.nn.relu(_ln(_dot(tok, w1_ref[0]) + b1_ref[0]))
    emb = _dot(h, w2_ref[0]) + b2_ref[0]
    out_ref[0] = _dot(emb, wtop_ref[...])


def _token_tables(tok3, w1s, b1s, w2s, b2s, wtop):
    out = pl.pallas_call(
        _tables_body,
        grid=(3,),
        in_specs=[
            pl.BlockSpec((1, _K, 8), lambda i: (i, 0, 0)),
            pl.BlockSpec((1, 8, _H), lambda i: (i, 0, 0)),
            pl.BlockSpec((1, 1, _H), lambda i: (i, 0, 0)),
            pl.BlockSpec((1, _H, _H), lambda i: (i, 0, 0)),
            pl.BlockSpec((1, 1, _H), lambda i: (i, 0, 0)),
            pl.BlockSpec((_H, _H), lambda i: (0, 0)),
        ],
        out_specs=pl.BlockSpec((1, _K, _H), lambda i: (i, 0, 0)),
        out_shape=jax.ShapeDtypeStruct((3, _K, _H), jnp.float32),
    )(tok3, w1s, b1s, w2s, b2s, wtop)
    return out.reshape(3 * _K, _H)


def _sc_gather(table, idx_pad):
    mesh = plsc.VectorSubcoreMesh(core_axis_name="c", subcore_axis_name="s")

    @functools.partial(
        pl.kernel,
        mesh=mesh,
        out_type=jax.ShapeDtypeStruct((_R, _H), jnp.float32),
        scratch_types=[
            pltpu.VMEM((_BPW,), jnp.int32),
            pltpu.VMEM((2, _CH, _H), jnp.float32),
            pltpu.SemaphoreType.DMA,
            pltpu.SemaphoreType.DMA,
        ],
    )
    def gk(table_hbm, idx_hbm, out_hbm, idx_v, rows_v, sem0, sem1):
        wid = lax.axis_index("s") * 2 + lax.axis_index("c")
        base = wid * _BPW
        pltpu.sync_copy(idx_hbm.at[pl.ds(base, _BPW)], idx_v)

        def live(j):
            return jnp.logical_and(j < _NCH, base + j * _CH < _R)

        def start(j, slot, sem):
            @pl.when(live(j))
            def _():
                pltpu.async_copy(
                    table_hbm.at[idx_v.at[pl.ds(j * _CH, _CH)]],
                    rows_v.at[slot], sem)

        def wait(j, slot, sem):
            @pl.when(live(j))
            def _():
                pltpu.make_async_copy(
                    table_hbm.at[idx_v.at[pl.ds(j * _CH, _CH)]],
                    rows_v.at[slot], sem).wait()

        def drain(j, slot):
            row0 = base + j * _CH

            @pl.when(row0 + _CH <= _R)
            def _():
                pltpu.sync_copy(rows_v.at[slot], out_hbm.at[pl.ds(row0, _CH)])

            @pl.when(jnp.logical_and(row0 < _R, row0 + _CH > _R))
            def _():
                pltpu.sync_copy(
                    rows_v.at[slot].at[pl.ds(0, _R % _CH)],
                    out_hbm.at[pl.ds(row0, _R % _CH)],
                )

        start(jnp.int32(0), 0, sem0)

        def body2(i, carry):
            j = i * 2
            start(j + 1, 1, sem1)
            wait(j, 0, sem0)
            drain(j, 0)
            start(j + 2, 0, sem0)
            wait(j + 1, 1, sem1)
            drain(j + 1, 1)
            return carry

        lax.fori_loop(0, _NCH // 2, body2, 0)

    return gk(table, idx_pad)


_BR = _BA * _S


def _main_body(px_ref, py_ref, hx_ref, hy_ref, shp_ref, oh_ref, gc_ref,
               f0_ref, f1_ref, wc0_ref, ws0_ref, wx0_ref, b10_ref,
               wc1_ref, ws1_ref, wx1_ref, b11_ref,
               w20_ref, b20_ref, w21_ref, b21_ref,
               outw_ref, outb_ref,
               sw1_ref, sb1_ref, sw2_ref, sb2_ref, te_ref,
               fwb_ref, fb1_ref, fw2_ref, fb2_ref,
               o_ref):
    px = px_ref[...]
    py = py_ref[...]
    mx = px - jnp.concatenate([px[:, :1], px[:, : _S - 1]], axis=1)
    my = py - jnp.concatenate([py[:, :1], py[:, : _S - 1]], axis=1)
    hx = hx_ref[...]
    hy = hy_ref[...]
    nrm = jnp.sqrt(mx * mx + my * my)
    ang = jnp.arctan2(hx * my - hy * mx, hx * mx + hy * my)
    ang = jnp.concatenate([jnp.zeros((_BA, 1), jnp.float32), ang[:, 1:]], axis=1)

    f0 = f0_ref[...]
    f1 = f1_ref[...]
    wc0, ws0, wx0 = wc0_ref[...], ws0_ref[...], wx0_ref[...]
    wc1, ws1, wx1 = wc1_ref[...], ws1_ref[...], wx1_ref[...]
    p0s, p1s = [], []
    for s in range(_S):
        nc = nrm[:, s : s + 1]
        ac = ang[:, s : s + 1]
        c0, s0 = _cossin2pi(nc * f0)
        c1, s1 = _cossin2pi(ac * f1)
        p0s.append(_dot(c0, wc0) + _dot(s0, ws0) + nc * wx0)
        p1s.append(_dot(c1, wc1) + _dot(s1, ws1) + ac * wx1)
    pre0 = jnp.concatenate(p0s, axis=0) + b10_ref[...]
    pre1 = jnp.concatenate(p1s, axis=0) + b11_ref[...]
    h0 = _dot(jax.nn.relu(_ln(pre0)), w20_ref[...]) + b20_ref[...]
    h1 = _dot(jax.nn.relu(_ln(pre1)), w21_ref[...]) + b21_ref[...]
    cat = _dot(
        jax.nn.relu(_ln(_dot(shp_ref[...], sw1_ref[...]) + sb1_ref[...])),
        sw2_ref[...],
    ) + sb2_ref[...] + _dot(oh_ref[...], te_ref[...])
    catr = jnp.concatenate([cat] * _S, axis=0)
    xa = _dot(jax.nn.relu(_ln(h0 + h1 + catr)), outw_ref[...]) + outb_ref[...]
    pre = gc_ref[...] + _dot(xa, fwb_ref[...]) + fb1_ref[...]
    feat = _dot(jax.nn.relu(_ln(pre)), fw2_ref[...]) + fb2_ref[...]
    for s in range(_S):
        o_ref[:, s, :] = feat[s * _BA : (s + 1) * _BA, :]


def _main(px, py, hx, hy, shp, oh, gc, weights):
    row = lambda: pl.BlockSpec((_BA, _S), lambda i: (i, 0))
    small = lambda a: pl.BlockSpec(a.shape, lambda i: (0,) * a.ndim)
    return pl.pallas_call(
        _main_body,
        grid=(_NB,),
        in_specs=[
            row(), row(), row(), row(),
            pl.BlockSpec((_BA, 3), lambda i: (i, 0)),
            pl.BlockSpec((_BA, 3), lambda i: (i, 0)),
            pl.BlockSpec((_BR, _H), lambda i: (i, 0)),
        ] + [small(w) for w in weights],
        out_specs=pl.BlockSpec((_BA, _S, _H), lambda i: (i, 0, 0)),
        out_shape=jax.ShapeDtypeStruct((_N, _S, _H), jnp.float32),
    )(px, py, hx, hy, shp, oh, gc, *weights)


def kernel(agent_token_index, trajectory_token_veh, trajectory_token_ped,
           trajectory_token_cyc, pos_a, head_vector_a, agent_type, agent_shape,
           type_emb_w, shape_W1, shape_b1, shape_W2, shape_b2,
           veh_W1, veh_b1, veh_W2, veh_b2, ped_W1, ped_b1, ped_W2, ped_b2,
           cyc_W1, cyc_b1, cyc_W2, cyc_b2, freqs, xa_W1, xa_b1, xa_W2, xa_b2,
           out_W, out_b, fus_W1, fus_b1, fus_W2, fus_b2):
    tok3 = jnp.stack([trajectory_token_veh, trajectory_token_ped,
                      trajectory_token_cyc])
    w1s = jnp.stack([veh_W1, ped_W1, cyc_W1])
    b1s = jnp.stack([veh_b1, ped_b1, cyc_b1])[:, None, :]
    w2s = jnp.stack([veh_W2, ped_W2, cyc_W2])
    b2s = jnp.stack([veh_b2, ped_b2, cyc_b2])[:, None, :]
    table = _token_tables(tok3, w1s, b1s, w2s, b2s, fus_W1[:_H])

    atype = agent_type.astype(jnp.int32)
    idx = (atype[:, None] * _K + agent_token_index.astype(jnp.int32)).reshape(-1)
    ii = jnp.arange(_NB, dtype=jnp.int32)[:, None, None]
    ss = jnp.arange(_S, dtype=jnp.int32)[None, :, None]
    aa = jnp.arange(_BA, dtype=jnp.int32)[None, None, :]
    orig = ((ii * _BA + aa) * _S + ss).reshape(-1)
    idx_pad = jnp.pad(idx[orig], (0, _RPAD - _R))
    gathered = _sc_gather(table, idx_pad)

    weights = (
        freqs[0:1],
        freqs[1:2],
        xa_W1[0, :_F, :], xa_W1[0, _F:2 * _F, :], xa_W1[0, 2 * _F:, :],
        xa_b1[0:1],
        xa_W1[1, :_F, :], xa_W1[1, _F:2 * _F, :], xa_W1[1, 2 * _F:, :],
        xa_b1[1:2],
        xa_W2[0], xa_b2[0:1], xa_W2[1], xa_b2[1:2],
        out_W, out_b[None, :],
        shape_W1, shape_b1[None, :], shape_W2, shape_b2[None, :], type_emb_w,
        fus_W1[_H:], fus_b1[None, :], fus_W2, fus_b2[None, :],
    )
    oh = jax.nn.one_hot(atype, 3, dtype=jnp.float32)
    return _main(pos_a[..., 0], pos_a[..., 1],
                 head_vector_a[..., 0], head_vector_a[..., 1],
                 agent_shape, oh, gathered, weights)


# speedup vs baseline: 1.0988x; 1.0019x over previous
import functools
import math

import jax
import jax.numpy as jnp
from jax import lax
from jax.experimental import pallas as pl
from jax.experimental.pallas import tpu as pltpu
from jax.experimental.pallas import tpu_sc as plsc

_N, _S, _H, _F, _K = 10000, 18, 128, 64, 2048
_R = _N * _S
_NW = 32
_BPW = 5632
_CH = 128
_NCH = _BPW // _CH
_RPAD = _NW * _BPW
_BA = 400
_NB = _N // _BA


def _ln(x):
    mu = jnp.mean(x, axis=-1, keepdims=True)
    xc = x - mu
    var = jnp.mean(xc * xc, axis=-1, keepdims=True)
    return xc * lax.rsqrt(var + 1e-5)


_CC = (1.0, -19.739208, 64.93939, -85.45669, 60.242466, -26.406763,
       7.8066154, -1.4609568)
_SS = (6.2831855, -41.3417, 81.60525, -76.70578, 42.057533, -15.085474,
       3.7785523, -0.6179781)
_MAGIC = 12582912.0


def _cossin2pi(y):
    r = y - jnp.round(y)
    r2 = r * r
    c = jnp.float32(_CC[-1])
    for a in _CC[-2::-1]:
        c = c * r2 + a
    s = jnp.float32(_SS[-1])
    for b in _SS[-2::-1]:
        s = s * r2 + b
    return c, s * r


def _dot(a, b):
    return jnp.dot(a.astype(jnp.bfloat16), b.astype(jnp.bfloat16),
                   preferred_element_type=jnp.float32)


def _tables_body(tok_ref, w1_ref, b1_ref, w2_ref, b2_ref, wtop_ref, out_ref):
    tok = tok_ref[0]
    h = jax.nn.relu(_ln(_dot(tok, w1_ref[0]) + b1_ref[0]))
    emb = _dot(h, w2_ref[0]) + b2_ref[0]
    out_ref[0] = _dot(emb, wtop_ref[...])


def _token_tables(tok3, w1s, b1s, w2s, b2s, wtop):
    out = pl.pallas_call(
        _tables_body,
        grid=(3,),
        in_specs=[
            pl.BlockSpec((1, _K, 8), lambda i: (i, 0, 0)),
            pl.BlockSpec((1, 8, _H), lambda i: (i, 0, 0)),
            pl.BlockSpec((1, 1, _H), lambda i: (i, 0, 0)),
            pl.BlockSpec((1, _H, _H), lambda i: (i, 0, 0)),
            pl.BlockSpec((1, 1, _H), lambda i: (i, 0, 0)),
            pl.BlockSpec((_H, _H), lambda i: (0, 0)),
        ],
        out_specs=pl.BlockSpec((1, _K, _H), lambda i: (i, 0, 0)),
        out_shape=jax.ShapeDtypeStruct((3, _K, _H), jnp.float32),
    )(tok3, w1s, b1s, w2s, b2s, wtop)
    return out.reshape(3 * _K, _H)


def _sc_gather(table, idx_pad):
    mesh = plsc.VectorSubcoreMesh(core_axis_name="c", subcore_axis_name="s")

    @functools.partial(
        pl.kernel,
        mesh=mesh,
        out_type=jax.ShapeDtypeStruct((_R, _H), jnp.float32),
        scratch_types=[
            pltpu.VMEM((_BPW,), jnp.int32),
            pltpu.VMEM((4, _CH, _H), jnp.float32),
            pltpu.SemaphoreType.DMA,
            pltpu.SemaphoreType.DMA,
            pltpu.SemaphoreType.DMA,
            pltpu.SemaphoreType.DMA,
        ],
    )
    def gk(table_hbm, idx_hbm, out_hbm, idx_v, rows_v, sem0, sem1, sem2, sem3):
        wid = lax.axis_index("s") * 2 + lax.axis_index("c")
        base = wid * _BPW
        pltpu.sync_copy(idx_hbm.at[pl.ds(base, _BPW)], idx_v)

        def live(j):
            return jnp.logical_and(j < _NCH, base + j * _CH < _R)

        def start(j, slot, sem):
            @pl.when(live(j))
            def _():
                pltpu.async_copy(
                    table_hbm.at[idx_v.at[pl.ds(j * _CH, _CH)]],
                    rows_v.at[slot], sem)

        def wait(j, slot, sem):
            @pl.when(live(j))
            def _():
                pltpu.make_async_copy(
                    table_hbm.at[idx_v.at[pl.ds(j * _CH, _CH)]],
                    rows_v.at[slot], sem).wait()

        def drain(j, slot):
            row0 = base + j * _CH

            @pl.when(row0 + _CH <= _R)
            def _():
                pltpu.sync_copy(rows_v.at[slot], out_hbm.at[pl.ds(row0, _CH)])

            @pl.when(jnp.logical_and(row0 < _R, row0 + _CH > _R))
            def _():
                pltpu.sync_copy(
                    rows_v.at[slot].at[pl.ds(0, _R % _CH)],
                    out_hbm.at[pl.ds(row0, _R % _CH)],
                )

        sems = (sem0, sem1, sem2, sem3)
        start(jnp.int32(0), 0, sem0)
        start(jnp.int32(1), 1, sem1)
        start(jnp.int32(2), 2, sem2)

        def body4(i, carry):
            j = i * 4
            for k in range(4):
                wait(j + k, k, sems[k])
                drain(j + k, k)
                nxt = (k + 3) % 4
                start(j + k + 3, nxt, sems[nxt])
            return carry

        lax.fori_loop(0, _NCH // 4, body4, 0)

    return gk(table, idx_pad)


_BR = _BA * _S


def _main_body(px_ref, py_ref, hx_ref, hy_ref, shp_ref, oh_ref, gc_ref,
               f0_ref, f1_ref, wc0_ref, ws0_ref, wx0_ref, b10_ref,
               wc1_ref, ws1_ref, wx1_ref, b11_ref,
               w20_ref, b20_ref, w21_ref, b21_ref,
               outw_ref, outb_ref,
               sw1_ref, sb1_ref, sw2_ref, sb2_ref, te_ref,
               fwb_ref, fb1_ref, fw2_ref, fb2_ref,
               o_ref):
    px = px_ref[...]
    py = py_ref[...]
    mx = px - jnp.concatenate([px[:, :1], px[:, : _S - 1]], axis=1)
    my = py - jnp.concatenate([py[:, :1], py[:, : _S - 1]], axis=1)
    hx = hx_ref[...]
    hy = hy_ref[...]
    nrm = jnp.sqrt(mx * mx + my * my)
    ang = jnp.arctan2(hx * my - hy * mx, hx * mx + hy * my)
    ang = jnp.concatenate([jnp.zeros((_BA, 1), jnp.float32), ang[:, 1:]], axis=1)

    f0 = f0_ref[...]
    f1 = f1_ref[...]
    wc0, ws0, wx0 = wc0_ref[...], ws0_ref[...], wx0_ref[...]
    wc1, ws1, wx1 = wc1_ref[...], ws1_ref[...], wx1_ref[...]
    p0s, p1s = [], []
    for s in range(_S):
        nc = nrm[:, s : s + 1]
        ac = ang[:, s : s + 1]
        c0, s0 = _cossin2pi(nc * f0)
        c1, s1 = _cossin2pi(ac * f1)
        p0s.append(_dot(c0, wc0) + _dot(s0, ws0) + nc * wx0)
        p1s.append(_dot(c1, wc1) + _dot(s1, ws1) + ac * wx1)
    pre0 = jnp.concatenate(p0s, axis=0) + b10_ref[...]
    pre1 = jnp.concatenate(p1s, axis=0) + b11_ref[...]
    h0 = _dot(jax.nn.relu(_ln(pre0)), w20_ref[...]) + b20_ref[...]
    h1 = _dot(jax.nn.relu(_ln(pre1)), w21_ref[...]) + b21_ref[...]
    cat = _dot(
        jax.nn.relu(_ln(_dot(shp_ref[...], sw1_ref[...]) + sb1_ref[...])),
        sw2_ref[...],
    ) + sb2_ref[...] + _dot(oh_ref[...], te_ref[...])
    catr = jnp.concatenate([cat] * _S, axis=0)
    xa = _dot(jax.nn.relu(_ln(h0 + h1 + catr)), outw_ref[...]) + outb_ref[...]
    pre = gc_ref[...] + _dot(xa, fwb_ref[...]) + fb1_ref[...]
    feat = _dot(jax.nn.relu(_ln(pre)), fw2_ref[...]) + fb2_ref[...]
    for s in range(_S):
        o_ref[:, s, :] = feat[s * _BA : (s + 1) * _BA, :]


def _main(px, py, hx, hy, shp, oh, gc, weights):
    row = lambda: pl.BlockSpec((_BA, _S), lambda i: (i, 0))
    small = lambda a: pl.BlockSpec(a.shape, lambda i: (0,) * a.ndim)
    return pl.pallas_call(
        _main_body,
        grid=(_NB,),
        in_specs=[
            row(), row(), row(), row(),
            pl.BlockSpec((_BA, 3), lambda i: (i, 0)),
            pl.BlockSpec((_BA, 3), lambda i: (i, 0)),
            pl.BlockSpec((_BR, _H), lambda i: (i, 0)),
        ] + [small(w) for w in weights],
        out_specs=pl.BlockSpec((_BA, _S, _H), lambda i: (i, 0, 0)),
        out_shape=jax.ShapeDtypeStruct((_N, _S, _H), jnp.float32),
    )(px, py, hx, hy, shp, oh, gc, *weights)


def kernel(agent_token_index, trajectory_token_veh, trajectory_token_ped,
           trajectory_token_cyc, pos_a, head_vector_a, agent_type, agent_shape,
           type_emb_w, shape_W1, shape_b1, shape_W2, shape_b2,
           veh_W1, veh_b1, veh_W2, veh_b2, ped_W1, ped_b1, ped_W2, ped_b2,
           cyc_W1, cyc_b1, cyc_W2, cyc_b2, freqs, xa_W1, xa_b1, xa_W2, xa_b2,
           out_W, out_b, fus_W1, fus_b1, fus_W2, fus_b2):
    tok3 = jnp.stack([trajectory_token_veh, trajectory_token_ped,
                      trajectory_token_cyc])
    w1s = jnp.stack([veh_W1, ped_W1, cyc_W1])
    b1s = jnp.stack([veh_b1, ped_b1, cyc_b1])[:, None, :]
    w2s = jnp.stack([veh_W2, ped_W2, cyc_W2])
    b2s = jnp.stack([veh_b2, ped_b2, cyc_b2])[:, None, :]
    table = _token_tables(tok3, w1s, b1s, w2s, b2s, fus_W1[:_H])

    atype = agent_type.astype(jnp.int32)
    idx = (atype[:, None] * _K + agent_token_index.astype(jnp.int32)).reshape(-1)
    ii = jnp.arange(_NB, dtype=jnp.int32)[:, None, None]
    ss = jnp.arange(_S, dtype=jnp.int32)[None, :, None]
    aa = jnp.arange(_BA, dtype=jnp.int32)[None, None, :]
    orig = ((ii * _BA + aa) * _S + ss).reshape(-1)
    idx_pad = jnp.pad(idx[orig], (0, _RPAD - _R))
    gathered = _sc_gather(table, idx_pad)

    weights = (
        freqs[0:1],
        freqs[1:2],
        xa_W1[0, :_F, :], xa_W1[0, _F:2 * _F, :], xa_W1[0, 2 * _F:, :],
        xa_b1[0:1],
        xa_W1[1, :_F, :], xa_W1[1, _F:2 * _F, :], xa_W1[1, 2 * _F:, :],
        xa_b1[1:2],
        xa_W2[0], xa_b2[0:1], xa_W2[1], xa_b2[1:2],
        out_W, out_b[None, :],
        shape_W1, shape_b1[None, :], shape_W2, shape_b2[None, :], type_emb_w,
        fus_W1[_H:], fus_b1[None, :], fus_W2, fus_b2[None, :],
    )
    oh = jax.nn.one_hot(atype, 3, dtype=jnp.float32)
    return _main(pos_a[..., 0], pos_a[..., 1],
                 head_vector_a[..., 0], head_vector_a[..., 1],
                 agent_shape, oh, gathered, weights)


# speedup vs baseline: 1.2007x; 1.0927x over previous
import functools
import math

import jax
import jax.numpy as jnp
from jax import lax
from jax.experimental import pallas as pl
from jax.experimental.pallas import tpu as pltpu
from jax.experimental.pallas import tpu_sc as plsc

_N, _S, _H, _F, _K = 10000, 18, 128, 64, 2048
_R = _N * _S
_NW = 32
_BPW = 5632
_CH = 128
_NCH = _BPW // _CH
_RPAD = _NW * _BPW
_BA = 400
_NB = _N // _BA


def _ln(x):
    mu = jnp.mean(x, axis=-1, keepdims=True)
    xc = x - mu
    var = jnp.mean(xc * xc, axis=-1, keepdims=True)
    return xc * lax.rsqrt(var + 1e-5)


_CC = (1.0, -19.739208, 64.93939, -85.45669, 60.242466, -26.406763,
       7.8066154, -1.4609568)
_SS = (6.2831855, -41.3417, 81.60525, -76.70578, 42.057533, -15.085474,
       3.7785523, -0.6179781)
_MAGIC = 12582912.0


def _cossin2pi(y):
    r = y - jnp.round(y)
    r2 = r * r
    c = jnp.float32(_CC[-1])
    for a in _CC[-2::-1]:
        c = c * r2 + a
    s = jnp.float32(_SS[-1])
    for b in _SS[-2::-1]:
        s = s * r2 + b
    return c, s * r


def _dot(a, b):
    return jnp.dot(a.astype(jnp.bfloat16), b.astype(jnp.bfloat16),
                   preferred_element_type=jnp.float32)


def _tables_body(tok_ref, w1_ref, b1_ref, w2_ref, b2_ref, wtop_ref, out_ref):
    tok = tok_ref[0]
    h = jax.nn.relu(_ln(_dot(tok, w1_ref[0]) + b1_ref[0]))
    emb = _dot(h, w2_ref[0]) + b2_ref[0]
    out_ref[0] = _dot(emb, wtop_ref[...])


def _token_tables(tok3, w1s, b1s, w2s, b2s, wtop):
    out = pl.pallas_call(
        _tables_body,
        grid=(3,),
        in_specs=[
            pl.BlockSpec((1, _K, 8), lambda i: (i, 0, 0)),
            pl.BlockSpec((1, 8, _H), lambda i: (i, 0, 0)),
            pl.BlockSpec((1, 1, _H), lambda i: (i, 0, 0)),
            pl.BlockSpec((1, _H, _H), lambda i: (i, 0, 0)),
            pl.BlockSpec((1, 1, _H), lambda i: (i, 0, 0)),
            pl.BlockSpec((_H, _H), lambda i: (0, 0)),
        ],
        out_specs=pl.BlockSpec((1, _K, _H), lambda i: (i, 0, 0)),
        out_shape=jax.ShapeDtypeStruct((3, _K, _H), jnp.float32),
    )(tok3, w1s, b1s, w2s, b2s, wtop)
    return out.reshape(3 * _K, _H)


def _sc_gather(table, idx_pad):
    mesh = plsc.VectorSubcoreMesh(core_axis_name="c", subcore_axis_name="s")

    @functools.partial(
        pl.kernel,
        mesh=mesh,
        out_type=jax.ShapeDtypeStruct((_R, _H), jnp.float32),
        scratch_types=[
            pltpu.VMEM((_BPW,), jnp.int32),
            pltpu.VMEM((4, _CH, _H), jnp.float32),
            pltpu.SemaphoreType.DMA,
            pltpu.SemaphoreType.DMA,
            pltpu.SemaphoreType.DMA,
            pltpu.SemaphoreType.DMA,
        ],
    )
    def gk(table_hbm, idx_hbm, out_hbm, idx_v, rows_v, sem0, sem1, sem2, sem3):
        wid = lax.axis_index("s") * 2 + lax.axis_index("c")
        base = wid * _BPW
        pltpu.sync_copy(idx_hbm.at[pl.ds(base, _BPW)], idx_v)

        def live(j):
            return jnp.logical_and(j < _NCH, base + j * _CH < _R)

        def start(j, slot, sem):
            @pl.when(live(j))
            def _():
                pltpu.async_copy(
                    table_hbm.at[idx_v.at[pl.ds(j * _CH, _CH)]],
                    rows_v.at[slot], sem)

        def wait(j, slot, sem):
            @pl.when(live(j))
            def _():
                pltpu.make_async_copy(
                    table_hbm.at[idx_v.at[pl.ds(j * _CH, _CH)]],
                    rows_v.at[slot], sem).wait()

        def drain(j, slot):
            row0 = base + j * _CH

            @pl.when(row0 + _CH <= _R)
            def _():
                pltpu.sync_copy(rows_v.at[slot], out_hbm.at[pl.ds(row0, _CH)])

            @pl.when(jnp.logical_and(row0 < _R, row0 + _CH > _R))
            def _():
                pltpu.sync_copy(
                    rows_v.at[slot].at[pl.ds(0, _R % _CH)],
                    out_hbm.at[pl.ds(row0, _R % _CH)],
                )

        sems = (sem0, sem1, sem2, sem3)
        start(jnp.int32(0), 0, sem0)
        start(jnp.int32(1), 1, sem1)
        start(jnp.int32(2), 2, sem2)

        def body4(i, carry):
            j = i * 4
            for k in range(4):
                wait(j + k, k, sems[k])
                drain(j + k, k)
                nxt = (k + 3) % 4
                start(j + k + 3, nxt, sems[nxt])
            return carry

        lax.fori_loop(0, _NCH // 4, body4, 0)

    return gk(table, idx_pad)


_BR = _BA * _S


def _main_body(px_ref, py_ref, hx_ref, hy_ref, shp_ref, oh_ref, gc_ref,
               f0_ref, f1_ref, wc0_ref, ws0_ref, wx0_ref, b10_ref,
               w20_ref, b20_ref,
               outw_ref, outb_ref,
               sw1_ref, sb1_ref, sw2_ref, sb2_ref, te_ref,
               fwb_ref, fb1_ref, fw2_ref, fb2_ref,
               o_ref):
    px = px_ref[...]
    py = py_ref[...]
    mx = px - jnp.concatenate([px[:, :1], px[:, : _S - 1]], axis=1)
    my = py - jnp.concatenate([py[:, :1], py[:, : _S - 1]], axis=1)
    hx = hx_ref[...]
    hy = hy_ref[...]
    nrm = jnp.sqrt(mx * mx + my * my)
    ang = jnp.arctan2(hx * my - hy * mx, hx * mx + hy * my)
    ang = jnp.concatenate([jnp.zeros((_BA, 1), jnp.float32), ang[:, 1:]], axis=1)

    f0 = f0_ref[...]
    f1 = f1_ref[...]
    wc01, ws01, wx01 = wc0_ref[...], ws0_ref[...], wx0_ref[...]
    ps = []
    for s in range(_S):
        nc = nrm[:, s : s + 1]
        ac = ang[:, s : s + 1]
        a01 = jnp.concatenate([nc * f0, ac * f1], axis=1)
        c01, s01 = _cossin2pi(a01)
        raw = jnp.concatenate([nc, ac], axis=1) @ wx01
        ps.append(_dot(c01, wc01) + _dot(s01, ws01) + raw)
    pre01 = jnp.concatenate(ps, axis=0) + b10_ref[...]
    z = jnp.concatenate(
        [jax.nn.relu(_ln(pre01[:, :_H])), jax.nn.relu(_ln(pre01[:, _H:]))],
        axis=1)
    h01 = _dot(z, w20_ref[...]) + b20_ref[...]
    cat = _dot(
        jax.nn.relu(_ln(_dot(shp_ref[...], sw1_ref[...]) + sb1_ref[...])),
        sw2_ref[...],
    ) + sb2_ref[...] + _dot(oh_ref[...], te_ref[...])
    catr = jnp.concatenate([cat] * _S, axis=0)
    xa = _dot(jax.nn.relu(_ln(h01 + catr)), outw_ref[...]) + outb_ref[...]
    pre = gc_ref[...] + _dot(xa, fwb_ref[...]) + fb1_ref[...]
    feat = _dot(jax.nn.relu(_ln(pre)), fw2_ref[...]) + fb2_ref[...]
    for s in range(_S):
        o_ref[:, s, :] = feat[s * _BA : (s + 1) * _BA, :]


def _main(px, py, hx, hy, shp, oh, gc, weights):
    row = lambda: pl.BlockSpec((_BA, _S), lambda i: (i, 0))
    small = lambda a: pl.BlockSpec(a.shape, lambda i: (0,) * a.ndim)
    return pl.pallas_call(
        _main_body,
        grid=(_NB,),
        in_specs=[
            row(), row(), row(), row(),
            pl.BlockSpec((_BA, 3), lambda i: (i, 0)),
            pl.BlockSpec((_BA, 3), lambda i: (i, 0)),
            pl.BlockSpec((_BR, _H), lambda i: (i, 0)),
        ] + [small(w) for w in weights],
        out_specs=pl.BlockSpec((_BA, _S, _H), lambda i: (i, 0, 0)),
        out_shape=jax.ShapeDtypeStruct((_N, _S, _H), jnp.float32),
    )(px, py, hx, hy, shp, oh, gc, *weights)


def kernel(agent_token_index, trajectory_token_veh, trajectory_token_ped,
           trajectory_token_cyc, pos_a, head_vector_a, agent_type, agent_shape,
           type_emb_w, shape_W1, shape_b1, shape_W2, shape_b2,
           veh_W1, veh_b1, veh_W2, veh_b2, ped_W1, ped_b1, ped_W2, ped_b2,
           cyc_W1, cyc_b1, cyc_W2, cyc_b2, freqs, xa_W1, xa_b1, xa_W2, xa_b2,
           out_W, out_b, fus_W1, fus_b1, fus_W2, fus_b2):
    tok3 = jnp.stack([trajectory_token_veh, trajectory_token_ped,
                      trajectory_token_cyc])
    w1s = jnp.stack([veh_W1, ped_W1, cyc_W1])
    b1s = jnp.stack([veh_b1, ped_b1, cyc_b1])[:, None, :]
    w2s = jnp.stack([veh_W2, ped_W2, cyc_W2])
    b2s = jnp.stack([veh_b2, ped_b2, cyc_b2])[:, None, :]
    table = _token_tables(tok3, w1s, b1s, w2s, b2s, fus_W1[:_H])

    atype = agent_type.astype(jnp.int32)
    idx = (atype[:, None] * _K + agent_token_index.astype(jnp.int32)).reshape(-1)
    ii = jnp.arange(_NB, dtype=jnp.int32)[:, None, None]
    ss = jnp.arange(_S, dtype=jnp.int32)[None, :, None]
    aa = jnp.arange(_BA, dtype=jnp.int32)[None, None, :]
    orig = ((ii * _BA + aa) * _S + ss).reshape(-1)
    idx_pad = jnp.pad(idx[orig], (0, _RPAD - _R))
    gathered = _sc_gather(table, idx_pad)

    zf = jnp.zeros((_F, _H), jnp.float32)
    wc01 = jnp.concatenate(
        [jnp.concatenate([xa_W1[0, :_F, :], zf], axis=1),
         jnp.concatenate([zf, xa_W1[1, :_F, :]], axis=1)], axis=0)
    ws01 = jnp.concatenate(
        [jnp.concatenate([xa_W1[0, _F:2 * _F, :], zf], axis=1),
         jnp.concatenate([zf, xa_W1[1, _F:2 * _F, :]], axis=1)], axis=0)
    zx = jnp.zeros((1, _H), jnp.float32)
    wx01 = jnp.concatenate(
        [jnp.concatenate([xa_W1[0, 2 * _F:, :], zx], axis=1),
         jnp.concatenate([zx, xa_W1[1, 2 * _F:, :]], axis=1)], axis=0)
    b01 = jnp.concatenate([xa_b1[0:1], xa_b1[1:2]], axis=1)
    w2s = jnp.concatenate([xa_W2[0], xa_W2[1]], axis=0)
    b2s = (xa_b2[0:1] + xa_b2[1:2])
    weights = (
        freqs[0:1],
        freqs[1:2],
        wc01, ws01, wx01, b01,
        w2s, b2s,
        out_W, out_b[None, :],
        shape_W1, shape_b1[None, :], shape_W2, shape_b2[None, :], type_emb_w,
        fus_W1[_H:], fus_b1[None, :], fus_W2, fus_b2[None, :],
    )
    oh = jax.nn.one_hot(atype, 3, dtype=jnp.float32)
    return _main(pos_a[..., 0], pos_a[..., 1],
                 head_vector_a[..., 0], head_vector_a[..., 1],
                 agent_shape, oh, gathered, weights)


# speedup vs baseline: 1.2232x; 1.0188x over previous
import functools
import math

import jax
import jax.numpy as jnp
from jax import lax
from jax.experimental import pallas as pl
from jax.experimental.pallas import tpu as pltpu
from jax.experimental.pallas import tpu_sc as plsc

_N, _S, _H, _F, _K = 10000, 18, 128, 64, 2048
_R = _N * _S
_NW = 32
_BPW = 5632
_CH = 128
_NCH = _BPW // _CH
_RPAD = _NW * _BPW
_BA = 400
_NB = _N // _BA


def _ln(x):
    mu = jnp.mean(x, axis=-1, keepdims=True)
    xc = x - mu
    var = jnp.mean(xc * xc, axis=-1, keepdims=True)
    return xc * lax.rsqrt(var + 1e-5)


_CC = (0.9999995, -19.739035, 64.93062, -85.29599, 58.912647, -21.283194)
_SS = (6.2831836, -41.34148, 81.597656, -76.594925, 41.269928, -12.372495)


def _cossin2pi(y):
    r = y - jnp.round(y)
    r2 = r * r
    c = jnp.float32(_CC[-1])
    for a in _CC[-2::-1]:
        c = c * r2 + a
    s = jnp.float32(_SS[-1])
    for b in _SS[-2::-1]:
        s = s * r2 + b
    return c, s * r


def _dot(a, b):
    return jnp.dot(a.astype(jnp.bfloat16), b.astype(jnp.bfloat16),
                   preferred_element_type=jnp.float32)


def _tables_body(tok_ref, w1_ref, b1_ref, w2_ref, b2_ref, wtop_ref, out_ref):
    tok = tok_ref[0]
    h = jax.nn.relu(_ln(_dot(tok, w1_ref[0]) + b1_ref[0]))
    emb = _dot(h, w2_ref[0]) + b2_ref[0]
    out_ref[0] = _dot(emb, wtop_ref[...])


def _token_tables(tok3, w1s, b1s, w2s, b2s, wtop):
    out = pl.pallas_call(
        _tables_body,
        grid=(3,),
        in_specs=[
            pl.BlockSpec((1, _K, 8), lambda i: (i, 0, 0)),
            pl.BlockSpec((1, 8, _H), lambda i: (i, 0, 0)),
            pl.BlockSpec((1, 1, _H), lambda i: (i, 0, 0)),
            pl.BlockSpec((1, _H, _H), lambda i: (i, 0, 0)),
            pl.BlockSpec((1, 1, _H), lambda i: (i, 0, 0)),
            pl.BlockSpec((_H, _H), lambda i: (0, 0)),
        ],
        out_specs=pl.BlockSpec((1, _K, _H), lambda i: (i, 0, 0)),
        out_shape=jax.ShapeDtypeStruct((3, _K, _H), jnp.float32),
    )(tok3, w1s, b1s, w2s, b2s, wtop)
    return out.reshape(3 * _K, _H)


def _sc_gather(table, idx_pad):
    mesh = plsc.VectorSubcoreMesh(core_axis_name="c", subcore_axis_name="s")

    @functools.partial(
        pl.kernel,
        mesh=mesh,
        out_type=jax.ShapeDtypeStruct((_R, _H), jnp.float32),
        scratch_types=[
            pltpu.VMEM((_BPW,), jnp.int32),
            pltpu.VMEM((4, _CH, _H), jnp.float32),
            pltpu.SemaphoreType.DMA,
            pltpu.SemaphoreType.DMA,
            pltpu.SemaphoreType.DMA,
            pltpu.SemaphoreType.DMA,
        ],
    )
    def gk(table_hbm, idx_hbm, out_hbm, idx_v, rows_v, sem0, sem1, sem2, sem3):
        wid = lax.axis_index("s") * 2 + lax.axis_index("c")
        base = wid * _BPW
        pltpu.sync_copy(idx_hbm.at[pl.ds(base, _BPW)], idx_v)

        def live(j):
            return jnp.logical_and(j < _NCH, base + j * _CH < _R)

        def start(j, slot, sem):
            @pl.when(live(j))
            def _():
                pltpu.async_copy(
                    table_hbm.at[idx_v.at[pl.ds(j * _CH, _CH)]],
                    rows_v.at[slot], sem)

        def wait(j, slot, sem):
            @pl.when(live(j))
            def _():
                pltpu.make_async_copy(
                    table_hbm.at[idx_v.at[pl.ds(j * _CH, _CH)]],
                    rows_v.at[slot], sem).wait()

        def drain(j, slot):
            row0 = base + j * _CH

            @pl.when(row0 + _CH <= _R)
            def _():
                pltpu.sync_copy(rows_v.at[slot], out_hbm.at[pl.ds(row0, _CH)])

            @pl.when(jnp.logical_and(row0 < _R, row0 + _CH > _R))
            def _():
                pltpu.sync_copy(
                    rows_v.at[slot].at[pl.ds(0, _R % _CH)],
                    out_hbm.at[pl.ds(row0, _R % _CH)],
                )

        sems = (sem0, sem1, sem2, sem3)
        start(jnp.int32(0), 0, sem0)
        start(jnp.int32(1), 1, sem1)
        start(jnp.int32(2), 2, sem2)

        def body4(i, carry):
            j = i * 4
            for k in range(4):
                wait(j + k, k, sems[k])
                drain(j + k, k)
                nxt = (k + 3) % 4
                start(j + k + 3, nxt, sems[nxt])
            return carry

        lax.fori_loop(0, _NCH // 4, body4, 0)

    return gk(table, idx_pad)


_BR = _BA * _S


def _main_body(px_ref, py_ref, hx_ref, hy_ref, shp_ref, oh_ref, gc_ref,
               f0_ref, f1_ref, wc0_ref, ws0_ref, wx0_ref, b10_ref,
               w20_ref, b20_ref,
               outw_ref, outb_ref,
               sw1_ref, sb1_ref, sw2_ref, sb2_ref, te_ref,
               fwb_ref, fb1_ref, fw2_ref, fb2_ref,
               o_ref):
    px = px_ref[...]
    py = py_ref[...]
    mx = px - jnp.concatenate([px[:, :1], px[:, : _S - 1]], axis=1)
    my = py - jnp.concatenate([py[:, :1], py[:, : _S - 1]], axis=1)
    hx = hx_ref[...]
    hy = hy_ref[...]
    nrm = jnp.sqrt(mx * mx + my * my)
    ang = jnp.arctan2(hx * my - hy * mx, hx * mx + hy * my)
    ang = jnp.concatenate([jnp.zeros((_BA, 1), jnp.float32), ang[:, 1:]], axis=1)

    f0 = f0_ref[...]
    f1 = f1_ref[...]
    wc01, ws01, wx01 = wc0_ref[...], ws0_ref[...], wx0_ref[...]
    ps = []
    for s in range(_S):
        nc = nrm[:, s : s + 1]
        ac = ang[:, s : s + 1]
        a01 = jnp.concatenate([nc * f0, ac * f1], axis=1)
        c01, s01 = _cossin2pi(a01)
        raw = jnp.concatenate([nc, ac], axis=1) @ wx01
        ps.append(_dot(c01, wc01) + _dot(s01, ws01) + raw)
    pre01 = jnp.concatenate(ps, axis=0) + b10_ref[...]
    z = jnp.concatenate(
        [jax.nn.relu(_ln(pre01[:, :_H])), jax.nn.relu(_ln(pre01[:, _H:]))],
        axis=1)
    h01 = _dot(z, w20_ref[...]) + b20_ref[...]
    cat = _dot(
        jax.nn.relu(_ln(_dot(shp_ref[...], sw1_ref[...]) + sb1_ref[...])),
        sw2_ref[...],
    ) + sb2_ref[...] + _dot(oh_ref[...], te_ref[...])
    catr = jnp.concatenate([cat] * _S, axis=0)
    xa = _dot(jax.nn.relu(_ln(h01 + catr)), outw_ref[...]) + outb_ref[...]
    pre = gc_ref[...] + _dot(xa, fwb_ref[...]) + fb1_ref[...]
    feat = _dot(jax.nn.relu(_ln(pre)), fw2_ref[...]) + fb2_ref[...]
    for s in range(_S):
        o_ref[:, s, :] = feat[s * _BA : (s + 1) * _BA, :]


def _main(px, py, hx, hy, shp, oh, gc, weights):
    row = lambda: pl.BlockSpec((_BA, _S), lambda i: (i, 0))
    small = lambda a: pl.BlockSpec(a.shape, lambda i: (0,) * a.ndim)
    return pl.pallas_call(
        _main_body,
        grid=(_NB,),
        in_specs=[
            row(), row(), row(), row(),
            pl.BlockSpec((_BA, 3), lambda i: (i, 0)),
            pl.BlockSpec((_BA, 3), lambda i: (i, 0)),
            pl.BlockSpec((_BR, _H), lambda i: (i, 0)),
        ] + [small(w) for w in weights],
        out_specs=pl.BlockSpec((_BA, _S, _H), lambda i: (i, 0, 0)),
        out_shape=jax.ShapeDtypeStruct((_N, _S, _H), jnp.float32),
    )(px, py, hx, hy, shp, oh, gc, *weights)


def kernel(agent_token_index, trajectory_token_veh, trajectory_token_ped,
           trajectory_token_cyc, pos_a, head_vector_a, agent_type, agent_shape,
           type_emb_w, shape_W1, shape_b1, shape_W2, shape_b2,
           veh_W1, veh_b1, veh_W2, veh_b2, ped_W1, ped_b1, ped_W2, ped_b2,
           cyc_W1, cyc_b1, cyc_W2, cyc_b2, freqs, xa_W1, xa_b1, xa_W2, xa_b2,
           out_W, out_b, fus_W1, fus_b1, fus_W2, fus_b2):
    tok3 = jnp.stack([trajectory_token_veh, trajectory_token_ped,
                      trajectory_token_cyc])
    w1s = jnp.stack([veh_W1, ped_W1, cyc_W1])
    b1s = jnp.stack([veh_b1, ped_b1, cyc_b1])[:, None, :]
    w2s = jnp.stack([veh_W2, ped_W2, cyc_W2])
    b2s = jnp.stack([veh_b2, ped_b2, cyc_b2])[:, None, :]
    table = _token_tables(tok3, w1s, b1s, w2s, b2s, fus_W1[:_H])

    atype = agent_type.astype(jnp.int32)
    idx = (atype[:, None] * _K + agent_token_index.astype(jnp.int32)).reshape(-1)
    ii = jnp.arange(_NB, dtype=jnp.int32)[:, None, None]
    ss = jnp.arange(_S, dtype=jnp.int32)[None, :, None]
    aa = jnp.arange(_BA, dtype=jnp.int32)[None, None, :]
    orig = ((ii * _BA + aa) * _S + ss).reshape(-1)
    idx_pad = jnp.pad(idx[orig], (0, _RPAD - _R))
    gathered = _sc_gather(table, idx_pad)

    zf = jnp.zeros((_F, _H), jnp.float32)
    wc01 = jnp.concatenate(
        [jnp.concatenate([xa_W1[0, :_F, :], zf], axis=1),
         jnp.concatenate([zf, xa_W1[1, :_F, :]], axis=1)], axis=0)
    ws01 = jnp.concatenate(
        [jnp.concatenate([xa_W1[0, _F:2 * _F, :], zf], axis=1),
         jnp.concatenate([zf, xa_W1[1, _F:2 * _F, :]], axis=1)], axis=0)
    zx = jnp.zeros((1, _H), jnp.float32)
    wx01 = jnp.concatenate(
        [jnp.concatenate([xa_W1[0, 2 * _F:, :], zx], axis=1),
         jnp.concatenate([zx, xa_W1[1, 2 * _F:, :]], axis=1)], axis=0)
    b01 = jnp.concatenate([xa_b1[0:1], xa_b1[1:2]], axis=1)
    w2s = jnp.concatenate([xa_W2[0], xa_W2[1]], axis=0)
    b2s = (xa_b2[0:1] + xa_b2[1:2])
    weights = (
        freqs[0:1],
        freqs[1:2],
        wc01, ws01, wx01, b01,
        w2s, b2s,
        out_W, out_b[None, :],
        shape_W1, shape_b1[None, :], shape_W2, shape_b2[None, :], type_emb_w,
        fus_W1[_H:], fus_b1[None, :], fus_W2, fus_b2[None, :],
    )
    oh = jax.nn.one_hot(atype, 3, dtype=jnp.float32)
    return _main(pos_a[..., 0], pos_a[..., 1],
                 head_vector_a[..., 0], head_vector_a[..., 1],
                 agent_shape, oh, gathered, weights)
